# baseline (speedup 1.0000x reference)
import jax
import jax.numpy as jnp
from jax import lax
from jax.experimental import pallas as pl
from jax.experimental.pallas import tpu as pltpu
from jax.experimental.pallas import tpu_sc as plsc

NC = 2
NS = 16
L = 16
NW = NC * NS

BATCH = 16384
EMB = 64
TR = 8
B_PER_W = BATCH // NW
RND = 32
NRND = B_PER_W // RND
GPR = RND // L


def _cf_body(user_ids_hbm, item_ids_hbm, user_table_hbm, item_table_hbm,
             item_bias_hbm, out_hbm,
             uidx_v, iidx_v, utile_v, itile_v, btile_v, out_v, sem):
    wid = lax.axis_index("s") * NC + lax.axis_index("c")
    base = wid * B_PER_W

    pltpu.sync_copy(user_ids_hbm.at[pl.ds(base, B_PER_W)], uidx_v)
    pltpu.sync_copy(item_ids_hbm.at[pl.ds(base, B_PER_W)], iidx_v)

    lanes = lax.iota(jnp.int32, L)
    zeros = jnp.zeros((L,), jnp.int32)

    def rnd(k, _):
        r0 = k * RND
        copies = []
        for t in range(GPR):
            uids = uidx_v[pl.ds(r0 + t * L, L)]
            iids = iidx_v[pl.ds(r0 + t * L, L)]
            ub = (uids >> 3) << 3
            ib = (iids >> 3) << 3
            for j in range(L):
                s = t * L + j
                ubj = pl.multiple_of(ub[j], TR)
                ibj = pl.multiple_of(ib[j], TR)
                copies.append(pltpu.async_copy(
                    user_table_hbm.at[pl.ds(ubj, TR), :],
                    utile_v.at[pl.ds(s * TR, TR), :], sem))
                copies.append(pltpu.async_copy(
                    item_table_hbm.at[pl.ds(ibj, TR), :],
                    itile_v.at[pl.ds(s * TR, TR), :], sem))
                copies.append(pltpu.async_copy(
                    item_bias_hbm.at[pl.ds(ibj, TR), :],
                    btile_v.at[pl.ds(s * TR, TR), :], sem))
        for cp in copies:
            cp.wait()

        for t in range(GPR):
            row16 = r0 + t * L + lanes
            uids = uidx_v[pl.ds(r0 + t * L, L)]
            iids = iidx_v[pl.ds(r0 + t * L, L)]
            slotbase = (t * L + lanes) * TR
            urow = slotbase + (uids & 7)
            irow = slotbase + (iids & 7)
            acc = plsc.load_gather(btile_v, [irow, zeros])
            for j in range(EMB):
                colj = jnp.full((L,), j, jnp.int32)
                u = plsc.load_gather(utile_v, [urow, colj])
                v = plsc.load_gather(itile_v, [irow, colj])
                acc = acc + u * v
            out_v[pl.ds(r0 + t * L, L)] = acc
        return 0

    lax.fori_loop(0, NRND, rnd, 0)

    pltpu.sync_copy(out_v, out_hbm.at[pl.ds(base, B_PER_W)])


@jax.jit
def kernel(user_ids, item_ids, user_table, item_table, item_bias):
    mesh = plsc.VectorSubcoreMesh(core_axis_name="c", subcore_axis_name="s")
    run = pl.kernel(
        _cf_body,
        out_type=jax.ShapeDtypeStruct((BATCH,), jnp.float32),
        mesh=mesh,
        scratch_types=[
            pltpu.VMEM((B_PER_W,), jnp.int32),
            pltpu.VMEM((B_PER_W,), jnp.int32),
            pltpu.VMEM((RND * TR, EMB), jnp.float32),
            pltpu.VMEM((RND * TR, EMB), jnp.float32),
            pltpu.VMEM((RND * TR, 1), jnp.float32),
            pltpu.VMEM((B_PER_W,), jnp.float32),
            pltpu.SemaphoreType.DMA,
        ],
        compiler_params=pltpu.CompilerParams(needs_layout_passes=False,
                                             use_tc_tiling_on_sc=True),
        name="cf_embedding_sc",
    )
    return run(user_ids.astype(jnp.int32), item_ids.astype(jnp.int32),
               user_table, item_table, item_bias)
